# baseline (speedup 1.0000x reference)
import functools

import jax
import jax.numpy as jnp
from jax import lax
from jax.experimental import pallas as pl
from jax.experimental.pallas import tpu as pltpu
from jax.experimental.pallas import tpu_sc as plsc

BATCH = 4096
MAXLEN = 200
EMBED = 64

_NC = 2
_NS = 16
_NW = _NC * _NS
_SEQ_PER_W = BATCH // _NW
_HALF = MAXLEN // 2
_VPR = EMBED // 16


def _body(x_hbm, tok_hbm, pos_hbm, out_hbm, idx_v, rows_v, pos_v, sem):
    wid = lax.axis_index("s") * _NC + lax.axis_index("c")
    seq0 = wid * _SEQ_PER_W

    pltpu.sync_copy(pos_hbm, pos_v)

    def per_seq(s, carry):
        gseq = seq0 + s
        pltpu.sync_copy(x_hbm.at[pl.ds(gseq * 2, 2)], idx_v)
        cp0 = pltpu.make_async_copy(
            tok_hbm.at[idx_v.at[0]], rows_v.at[pl.ds(0, _HALF)], sem)
        cp1 = pltpu.make_async_copy(
            tok_hbm.at[idx_v.at[1]], rows_v.at[pl.ds(_HALF, _HALF)], sem)
        cp0.start()
        cp1.start()
        cp0.wait()
        cp1.wait()

        def add_row(r, c):
            for j in range(_VPR):
                sl = pl.ds(j * 16, 16)
                rows_v[r, sl] = rows_v[r, sl] + pos_v[r, sl]
            return c

        lax.fori_loop(0, MAXLEN, add_row, 0, unroll=2)

        pltpu.sync_copy(rows_v, out_hbm.at[pl.ds(gseq * MAXLEN, MAXLEN)])
        return carry

    lax.fori_loop(0, _SEQ_PER_W, per_seq, 0)


@jax.jit
def _tpe(x2, token_table, pos_table):
    mesh = plsc.VectorSubcoreMesh(core_axis_name="c", subcore_axis_name="s")
    return pl.kernel(
        _body,
        out_type=jax.ShapeDtypeStruct((BATCH * MAXLEN, EMBED), jnp.float32),
        mesh=mesh,
        scratch_types=[
            pltpu.VMEM((2, _HALF), jnp.int32),
            pltpu.VMEM((MAXLEN, EMBED), jnp.float32),
            pltpu.VMEM((MAXLEN, EMBED), jnp.float32),
            pltpu.SemaphoreType.DMA,
        ],
        compiler_params=pltpu.CompilerParams(use_tc_tiling_on_sc=False),
    )(x2, token_table, pos_table)


def kernel(x, token_table, pos_table):
    x2 = x.astype(jnp.int32).reshape(BATCH * 2, _HALF)
    out = _tpe(x2, token_table, pos_table)
    return out.reshape(BATCH, MAXLEN, EMBED)
